# Initial kernel scaffold; baseline (speedup 1.0000x reference)
import jax
import jax.numpy as jnp
from jax.experimental import pallas as pl


def kernel(sen_ids, table):
    raise NotImplementedError("write your pallas kernel here")


# speedup vs baseline: 3.5428x; 3.5428x over previous
import functools

import jax
import jax.numpy as jnp
from jax import lax
from jax.experimental import pallas as pl
from jax.experimental.pallas import tpu as pltpu
from jax.experimental.pallas import tpu_sc as plsc

_NC = 2
_NS = 16
_NW = _NC * _NS

_VOCAB = 100000
_D = 64
_CHUNK = 128


def _make_lookup(B):
    assert B % (_NW * _CHUNK) == 0
    per_w = B // _NW
    nch = per_w // _CHUNK
    mesh = plsc.VectorSubcoreMesh(core_axis_name="c", subcore_axis_name="s")

    @functools.partial(
        pl.kernel,
        out_type=jax.ShapeDtypeStruct((B, _D), jnp.float32),
        mesh=mesh,
        scratch_types=[
            pltpu.VMEM((nch, _CHUNK), jnp.int32),
            pltpu.VMEM((_CHUNK, _D), jnp.float32),
            pltpu.SemaphoreType.DMA,
        ],
        compiler_params=pltpu.CompilerParams(use_tc_tiling_on_sc=False),
    )
    def lookup(table_hbm, idx_hbm, out_hbm, idx_v, rows_v, sem):
        wid = lax.axis_index("s") * _NC + lax.axis_index("c")
        base = pl.multiple_of(wid * per_w, _CHUNK)
        pltpu.sync_copy(idx_hbm.at[wid], idx_v)

        def body(j, _):
            off = pl.multiple_of(base + j * _CHUNK, _CHUNK)
            pltpu.async_copy(table_hbm.at[idx_v.at[j]], rows_v, sem).wait()
            pltpu.sync_copy(rows_v, out_hbm.at[pl.ds(off, _CHUNK)])
            return 0

        lax.fori_loop(0, nch, body, 0)

    return lookup


def kernel(sen_ids, table):
    S, T = sen_ids.shape
    B = S * T
    idx = sen_ids.reshape(-1).astype(jnp.int32)
    idx3 = idx.reshape(_NW, B // (_NW * _CHUNK), _CHUNK)
    out = _make_lookup(B)(table, idx3)
    return out.reshape(S, T, _D)


# speedup vs baseline: 4.2537x; 1.2007x over previous
import functools

import jax
import jax.numpy as jnp
from jax import lax
from jax.experimental import pallas as pl
from jax.experimental.pallas import tpu as pltpu
from jax.experimental.pallas import tpu_sc as plsc

_NC = 2
_NS = 16
_NW = _NC * _NS

_D = 64
_CHUNK = 128
_G = 4


def _make_lookup(B):
    rows_per_group = _G * _CHUNK
    assert B % (_NW * rows_per_group) == 0
    per_w = B // _NW
    ngroups = per_w // rows_per_group
    assert ngroups % 2 == 0 and ngroups >= 4
    mesh = plsc.VectorSubcoreMesh(core_axis_name="c", subcore_axis_name="s")

    @functools.partial(
        pl.kernel,
        out_type=jax.ShapeDtypeStruct((B, _D), jnp.float32),
        mesh=mesh,
        scratch_types=[
            pltpu.VMEM((per_w // _CHUNK, _CHUNK), jnp.int32),
            pltpu.VMEM((rows_per_group, _D), jnp.float32),
            pltpu.VMEM((rows_per_group, _D), jnp.float32),
            pltpu.SemaphoreType.DMA,
            pltpu.SemaphoreType.DMA,
        ],
        compiler_params=pltpu.CompilerParams(use_tc_tiling_on_sc=False),
    )
    def lookup(table_hbm, idx_hbm, out_hbm, idx_v, rows0, rows1, sem0, sem1):
        wid = lax.axis_index("s") * _NC + lax.axis_index("c")
        base = pl.multiple_of(wid * per_w, _CHUNK)
        pltpu.sync_copy(idx_hbm.at[wid], idx_v)

        bufs = (rows0, rows1)
        sems = (sem0, sem1)

        def fire(t, b):
            for i in range(_G):
                pltpu.async_copy(
                    table_hbm.at[idx_v.at[t * _G + i]],
                    bufs[b].at[pl.ds(i * _CHUNK, _CHUNK)],
                    sems[b],
                )

        def drain(b):
            for i in range(_G):
                pltpu.make_async_copy(
                    table_hbm.at[idx_v.at[i]],
                    bufs[b].at[pl.ds(i * _CHUNK, _CHUNK)],
                    sems[b],
                ).wait()

        def copy_out(t, b):
            off = pl.multiple_of(base + t * rows_per_group, _CHUNK)
            pltpu.sync_copy(bufs[b], out_hbm.at[pl.ds(off, rows_per_group)])

        fire(0, 0)

        def body(i0, _):
            t0 = 2 * i0
            fire(t0 + 1, 1)
            drain(0)
            copy_out(t0, 0)
            fire(t0 + 2, 0)
            drain(1)
            copy_out(t0 + 1, 1)
            return 0

        lax.fori_loop(0, (ngroups - 2) // 2, body, 0)

        fire(ngroups - 1, 1)
        drain(0)
        copy_out(ngroups - 2, 0)
        drain(1)
        copy_out(ngroups - 1, 1)

    return lookup


def kernel(sen_ids, table):
    S, T = sen_ids.shape
    B = S * T
    idx = sen_ids.reshape(-1).astype(jnp.int32)
    idx3 = idx.reshape(_NW, B // (_NW * _CHUNK), _CHUNK)
    out = _make_lookup(B)(table, idx3)
    return out.reshape(S, T, _D)
